# Initial kernel scaffold; baseline (speedup 1.0000x reference)
import jax
import jax.numpy as jnp
from jax.experimental import pallas as pl


def kernel(fatoms, fbonds, agraph, bgraph, scope, W_i, W_h, W_o, b_o):
    raise NotImplementedError("write your pallas kernel here")


# speedup vs baseline: 2.7226x; 2.7226x over previous
import functools

import jax
import jax.numpy as jnp
from jax import lax
from jax.experimental import pallas as pl
from jax.experimental.pallas import tpu as pltpu
from jax.experimental.pallas import tpu_sc as plsc

_HID = 128
_NW = 32
_C = 128
_DEPTH = 4
_MLEN = 25

_BR_B = 1024
_BR_A = 3200


def _make_gather_sum(bp):
    cpw = bp // _NW
    nchunk = cpw // _C
    mesh = plsc.VectorSubcoreMesh(core_axis_name="c", subcore_axis_name="s")

    @functools.partial(
        pl.kernel,
        mesh=mesh,
        out_type=jax.ShapeDtypeStruct((bp, _HID), jnp.float32),
        scratch_types=[
            pltpu.VMEM((6, _C), jnp.int32),
            pltpu.VMEM((6, _C, _HID), jnp.float32),
            pltpu.VMEM((_C, _HID), jnp.float32),
            pltpu.SemaphoreType.DMA,
        ],
    )
    def gather_sum(table_hbm, idx_hbm, out_hbm, idx_v, bufs_v, out_v, sem):
        cid = lax.axis_index("c")
        sid = lax.axis_index("s")
        wid = sid * 2 + cid
        base = wid * cpw

        def chunk(ci, carry):
            pltpu.sync_copy(idx_hbm.at[wid, ci], idx_v)
            cps = []
            for j in range(6):
                cps.append(
                    pltpu.async_copy(table_hbm.at[idx_v.at[j]], bufs_v.at[j], sem)
                )
            for cp in cps:
                cp.wait()

            def row(r, c2):
                for l in range(8):
                    s = pl.ds(l * 16, 16)
                    acc = bufs_v[0, r, s] + bufs_v[1, r, s]
                    acc = acc + bufs_v[2, r, s]
                    acc = acc + bufs_v[3, r, s]
                    acc = acc + bufs_v[4, r, s]
                    acc = acc + bufs_v[5, r, s]
                    out_v[r, s] = acc
                return c2

            lax.fori_loop(0, _C, row, 0)
            pltpu.sync_copy(out_v, out_hbm.at[pl.ds(base + ci * _C, _C)])
            return carry

        lax.fori_loop(0, nchunk, chunk, 0)

    return gather_sum


def _prep_idx(g, bp):
    b = g.shape[0]
    cpw = bp // _NW
    nchunk = cpw // _C
    gi = g.astype(jnp.int32)
    gi = jnp.pad(gi, ((0, bp - b), (0, 0)))
    gi = gi.reshape(_NW, nchunk, _C, 6)
    return gi.transpose(0, 1, 3, 2)


def _nt(x, w):
    return lax.dot_general(x, w, (((1,), (1,)), ((), ())),
                           preferred_element_type=jnp.float32)


def _bonds_input_body(x_ref, w_ref, bin_ref, msg_ref):
    b = _nt(x_ref[...], w_ref[...])
    bin_ref[...] = b
    msg_ref[...] = jnp.maximum(b, 0.0)


def _bonds_input(fbonds, w_i, bp):
    n, k = fbonds.shape
    grid = bp // _BR_B
    return pl.pallas_call(
        _bonds_input_body,
        grid=(grid,),
        in_specs=[
            pl.BlockSpec((_BR_B, k), lambda i: (i, 0)),
            pl.BlockSpec((_HID, k), lambda i: (0, 0)),
        ],
        out_specs=[
            pl.BlockSpec((_BR_B, _HID), lambda i: (i, 0)),
            pl.BlockSpec((_BR_B, _HID), lambda i: (i, 0)),
        ],
        out_shape=[
            jax.ShapeDtypeStruct((bp, _HID), jnp.float32),
            jax.ShapeDtypeStruct((bp, _HID), jnp.float32),
        ],
    )(fbonds, w_i)


def _iter_body(nei_ref, bin_ref, w_ref, out_ref):
    h = _nt(nei_ref[...], w_ref[...])
    out_ref[...] = jnp.maximum(bin_ref[...] + h, 0.0)


def _iter_step(nei, binput, w_h):
    bp = nei.shape[0]
    grid = bp // _BR_B
    return pl.pallas_call(
        _iter_body,
        grid=(grid,),
        in_specs=[
            pl.BlockSpec((_BR_B, _HID), lambda i: (i, 0)),
            pl.BlockSpec((_BR_B, _HID), lambda i: (i, 0)),
            pl.BlockSpec((_HID, _HID), lambda i: (0, 0)),
        ],
        out_specs=pl.BlockSpec((_BR_B, _HID), lambda i: (i, 0)),
        out_shape=jax.ShapeDtypeStruct((bp, _HID), jnp.float32),
    )(nei, binput, w_h)


def _final_body(n_valid, fa_ref, nei_ref, wa_ref, wn_ref, bo_ref, out_ref):
    h = _nt(fa_ref[...], wa_ref[...]) + _nt(nei_ref[...], wn_ref[...])
    h = jnp.maximum(h + bo_ref[...], 0.0)
    row = (lax.broadcasted_iota(jnp.int32, (_BR_A, _HID), 0)
           + pl.program_id(0) * _BR_A)
    h = jnp.where(row < n_valid, h, 0.0)
    nm = _BR_A // _MLEN
    am = lax.broadcasted_iota(jnp.int32, (nm, _BR_A), 1) // _MLEN
    mm = lax.broadcasted_iota(jnp.int32, (nm, _BR_A), 0)
    pool = (am == mm).astype(jnp.float32)
    out_ref[...] = lax.dot_general(pool, h, (((1,), (0,)), ((), ())),
                                   preferred_element_type=jnp.float32)


def _final_stage(fatoms, nei_a, w_oa, w_on, b_o2d, ap):
    n, fa = fatoms.shape
    grid = ap // _BR_A
    nm = _BR_A // _MLEN
    return pl.pallas_call(
        functools.partial(_final_body, n),
        grid=(grid,),
        in_specs=[
            pl.BlockSpec((_BR_A, fa), lambda i: (i, 0)),
            pl.BlockSpec((_BR_A, _HID), lambda i: (i, 0)),
            pl.BlockSpec((_HID, fa), lambda i: (0, 0)),
            pl.BlockSpec((_HID, _HID), lambda i: (0, 0)),
            pl.BlockSpec((1, _HID), lambda i: (0, 0)),
        ],
        out_specs=pl.BlockSpec((nm, _HID), lambda i: (i, 0)),
        out_shape=jax.ShapeDtypeStruct((grid * nm, _HID), jnp.float32),
    )(fatoms, nei_a, w_oa, w_on, b_o2d)


def kernel(fatoms, fbonds, agraph, bgraph, scope, W_i, W_h, W_o, b_o):
    n_atoms, atom_fdim = fatoms.shape
    n_bonds = fbonds.shape[0]
    n_mols = scope.shape[0]

    bp = 200704
    ap = 102400

    bg_idx = _prep_idx(bgraph, bp)
    ag_idx = _prep_idx(agraph, ap)

    binput, message = _bonds_input(fbonds, W_i, bp)
    gather_b = _make_gather_sum(bp)
    for _ in range(_DEPTH - 1):
        nei = gather_b(message, bg_idx)
        message = _iter_step(nei, binput, W_h)

    gather_a = _make_gather_sum(ap)
    nei_a = gather_a(message, ag_idx)

    w_oa = W_o[:, :atom_fdim]
    w_on = W_o[:, atom_fdim:]
    mol_sums = _final_stage(fatoms, nei_a, w_oa, w_on,
                            b_o.reshape(1, _HID), ap)
    lengths = scope[:, 1].astype(jnp.float32)
    return mol_sums[:n_mols] / lengths[:, None]


# speedup vs baseline: 3.7575x; 1.3801x over previous
import functools

import jax
import jax.numpy as jnp
from jax import lax
from jax.experimental import pallas as pl
from jax.experimental.pallas import tpu as pltpu
from jax.experimental.pallas import tpu_sc as plsc

_HID = 128
_NW = 32
_C = 64
_DEPTH = 4
_MLEN = 25

_BR_B = 1024
_BR_A = 3200


def _make_gather_sum(bp):
    cpw = bp // _NW
    nchunk = cpw // _C
    mesh = plsc.VectorSubcoreMesh(core_axis_name="c", subcore_axis_name="s")

    @functools.partial(
        pl.kernel,
        mesh=mesh,
        out_type=jax.ShapeDtypeStruct((bp, _HID), jnp.float32),
        scratch_types=[
            pltpu.VMEM((2, 6, _C), jnp.int32),
            pltpu.VMEM((2, 6, _C, _HID), jnp.float32),
            pltpu.VMEM((2, _C, _HID), jnp.float32),
            pltpu.SemaphoreType.DMA,
            pltpu.SemaphoreType.DMA,
        ],
    )
    def gather_sum(table_hbm, idx_hbm, out_hbm, idx_v, bufs_v, out_v,
                   sem_a, sem_b):
        cid = lax.axis_index("c")
        sid = lax.axis_index("s")
        wid = sid * 2 + cid
        base = wid * cpw
        sems = (sem_a, sem_b)

        def fire(ci, s):
            pltpu.sync_copy(idx_hbm.at[wid, ci], idx_v.at[s])
            for j in range(6):
                pltpu.async_copy(table_hbm.at[idx_v.at[s, j]],
                                 bufs_v.at[s, j], sems[s])

        def drain(s):
            for j in range(6):
                pltpu.make_async_copy(table_hbm.at[idx_v.at[s, j]],
                                      bufs_v.at[s, j], sems[s]).wait()

        def add(s):
            def row(r, c2):
                for l in range(8):
                    sl = pl.ds(l * 16, 16)
                    acc = bufs_v[s, 0, r, sl] + bufs_v[s, 1, r, sl]
                    acc = acc + bufs_v[s, 2, r, sl]
                    acc = acc + bufs_v[s, 3, r, sl]
                    acc = acc + bufs_v[s, 4, r, sl]
                    acc = acc + bufs_v[s, 5, r, sl]
                    out_v[s, r, sl] = acc
                return c2

            lax.fori_loop(0, _C, row, 0)

        fire(0, 0)
        fire(1, 1)

        def pair(p, carry):
            c0 = 2 * p
            for s in range(2):
                drain(s)
                add(s)

                @pl.when(c0 + 2 + s < nchunk)
                def _():
                    fire(c0 + 2 + s, s)

                pltpu.sync_copy(out_v.at[s],
                                out_hbm.at[pl.ds(base + (c0 + s) * _C, _C)])
            return carry

        lax.fori_loop(0, nchunk // 2, pair, 0)

    return gather_sum


def _prep_idx(g, bp):
    b = g.shape[0]
    cpw = bp // _NW
    nchunk = cpw // _C
    gi = g.astype(jnp.int32)
    gi = jnp.pad(gi, ((0, bp - b), (0, 0)))
    gi = gi.reshape(_NW, nchunk, _C, 6)
    return gi.transpose(0, 1, 3, 2)


def _nt(x, w):
    return lax.dot_general(x, w, (((1,), (1,)), ((), ())),
                           preferred_element_type=jnp.float32)


def _bonds_input_body(x_ref, w_ref, bin_ref, msg_ref):
    b = _nt(x_ref[...], w_ref[...])
    bin_ref[...] = b
    msg_ref[...] = jnp.maximum(b, 0.0)


def _bonds_input(fbonds, w_i, bp):
    n, k = fbonds.shape
    grid = bp // _BR_B
    return pl.pallas_call(
        _bonds_input_body,
        grid=(grid,),
        in_specs=[
            pl.BlockSpec((_BR_B, k), lambda i: (i, 0)),
            pl.BlockSpec((_HID, k), lambda i: (0, 0)),
        ],
        out_specs=[
            pl.BlockSpec((_BR_B, _HID), lambda i: (i, 0)),
            pl.BlockSpec((_BR_B, _HID), lambda i: (i, 0)),
        ],
        out_shape=[
            jax.ShapeDtypeStruct((bp, _HID), jnp.float32),
            jax.ShapeDtypeStruct((bp, _HID), jnp.float32),
        ],
    )(fbonds, w_i)


def _iter_body(nei_ref, bin_ref, w_ref, out_ref):
    h = _nt(nei_ref[...], w_ref[...])
    out_ref[...] = jnp.maximum(bin_ref[...] + h, 0.0)


def _iter_step(nei, binput, w_h):
    bp = nei.shape[0]
    grid = bp // _BR_B
    return pl.pallas_call(
        _iter_body,
        grid=(grid,),
        in_specs=[
            pl.BlockSpec((_BR_B, _HID), lambda i: (i, 0)),
            pl.BlockSpec((_BR_B, _HID), lambda i: (i, 0)),
            pl.BlockSpec((_HID, _HID), lambda i: (0, 0)),
        ],
        out_specs=pl.BlockSpec((_BR_B, _HID), lambda i: (i, 0)),
        out_shape=jax.ShapeDtypeStruct((bp, _HID), jnp.float32),
    )(nei, binput, w_h)


def _final_body(n_valid, fa_ref, nei_ref, wa_ref, wn_ref, bo_ref, out_ref):
    h = _nt(fa_ref[...], wa_ref[...]) + _nt(nei_ref[...], wn_ref[...])
    h = jnp.maximum(h + bo_ref[...], 0.0)
    row = (lax.broadcasted_iota(jnp.int32, (_BR_A, _HID), 0)
           + pl.program_id(0) * _BR_A)
    h = jnp.where(row < n_valid, h, 0.0)
    nm = _BR_A // _MLEN
    am = lax.broadcasted_iota(jnp.int32, (nm, _BR_A), 1) // _MLEN
    mm = lax.broadcasted_iota(jnp.int32, (nm, _BR_A), 0)
    pool = (am == mm).astype(jnp.float32)
    out_ref[...] = lax.dot_general(pool, h, (((1,), (0,)), ((), ())),
                                   preferred_element_type=jnp.float32)


def _final_stage(fatoms, nei_a, w_oa, w_on, b_o2d, ap):
    n, fa = fatoms.shape
    grid = ap // _BR_A
    nm = _BR_A // _MLEN
    return pl.pallas_call(
        functools.partial(_final_body, n),
        grid=(grid,),
        in_specs=[
            pl.BlockSpec((_BR_A, fa), lambda i: (i, 0)),
            pl.BlockSpec((_BR_A, _HID), lambda i: (i, 0)),
            pl.BlockSpec((_HID, fa), lambda i: (0, 0)),
            pl.BlockSpec((_HID, _HID), lambda i: (0, 0)),
            pl.BlockSpec((1, _HID), lambda i: (0, 0)),
        ],
        out_specs=pl.BlockSpec((nm, _HID), lambda i: (i, 0)),
        out_shape=jax.ShapeDtypeStruct((grid * nm, _HID), jnp.float32),
    )(fatoms, nei_a, w_oa, w_on, b_o2d)


def kernel(fatoms, fbonds, agraph, bgraph, scope, W_i, W_h, W_o, b_o):
    n_atoms, atom_fdim = fatoms.shape
    n_bonds = fbonds.shape[0]
    n_mols = scope.shape[0]

    bp = 200704
    ap = 102400

    bg_idx = _prep_idx(bgraph, bp)
    ag_idx = _prep_idx(agraph, ap)

    binput, message = _bonds_input(fbonds, W_i, bp)
    gather_b = _make_gather_sum(bp)
    for _ in range(_DEPTH - 1):
        nei = gather_b(message, bg_idx)
        message = _iter_step(nei, binput, W_h)

    gather_a = _make_gather_sum(ap)
    nei_a = gather_a(message, ag_idx)

    w_oa = W_o[:, :atom_fdim]
    w_on = W_o[:, atom_fdim:]
    mol_sums = _final_stage(fatoms, nei_a, w_oa, w_on,
                            b_o.reshape(1, _HID), ap)
    lengths = scope[:, 1].astype(jnp.float32)
    return mol_sums[:n_mols] / lengths[:, None]
